# baseline (speedup 1.0000x reference)
import functools

import jax
import jax.numpy as jnp
from jax import lax
from jax.experimental import pallas as pl
from jax.experimental.pallas import tpu as pltpu
from jax.experimental.pallas import tpu_sc as plsc

B = 4096
L = 200
LP = 208
NBATCH = LP // 16
NGROUP = 4
D = 16
NC, NS = 2, 16
NW = NC * NS
NPW = B // NW

_f32 = jnp.float32
_i32 = jnp.int32


def _iota16():
    return lax.iota(_i32, 16)


def _splat_i(v):
    return jnp.full((16,), v, _i32)


def _sc_body(hist_uv, hist_rf, nodes, v2e, u2e,
             w1e, c1r, w2e, w2be, a1e, a1uc, a1b, a2e, a2be, a3e,
             out,
             huv_v, rv1_v, nidx_v, rows2_v, uvrows_v,
             w1e_v, c1r_v, w2e_v, w2be_v, a1e_v, a1uc_v, a1b_v,
             a2e_v, a2be_v, a3e_v,
             ot_v, cvec_v, tb_v, outb_v, sems, sem):
    wid = lax.axis_index("s") * NC + lax.axis_index("c")
    base_node = wid * NPW

    pltpu.sync_copy(w1e, w1e_v)
    pltpu.sync_copy(c1r, c1r_v)
    pltpu.sync_copy(w2e, w2e_v)
    pltpu.sync_copy(w2be, w2be_v)
    pltpu.sync_copy(a1e, a1e_v)
    pltpu.sync_copy(a1uc, a1uc_v)
    pltpu.sync_copy(a1b, a1b_v)
    pltpu.sync_copy(a2e, a2e_v)
    pltpu.sync_copy(a2be, a2be_v)
    pltpu.sync_copy(a3e, a3e_v)

    rv1_v[pl.ds(NPW * L - 8, 16)] = jnp.zeros((16,), _i32)
    pltpu.sync_copy(hist_uv.at[pl.ds(base_node, NPW)], huv_v)
    pltpu.sync_copy(hist_rf.at[pl.ds(base_node * L, NPW * L)],
                    rv1_v.at[pl.ds(0, NPW * L)])

    pltpu.sync_copy(nodes.at[pl.ds(base_node, NPW)], nidx_v)
    pltpu.async_copy(u2e.at[nidx_v], uvrows_v, sem).wait()

    def catt_body(n, _):
        catt = a1b_v[...]
        for k in range(D):
            uk = plsc.load_gather(uvrows_v, [_splat_i(n), _splat_i(k)])
            catt = catt + uk * a1uc_v[k]
        cvec_v[n] = catt
        return 0

    lax.fori_loop(0, NPW, catt_body, 0, unroll=False)

    for s in range(2):
        for r in range(L, LP):
            rows2_v[s, r] = jnp.zeros((16,), _f32)

    pltpu.async_copy(v2e.at[huv_v.at[0]], rows2_v.at[0, pl.ds(0, L)],
                     sems.at[0])

    def node_body(n, _):
        sel = lax.rem(n, 2)
        nsel = 1 - sel
        pltpu.make_async_copy(v2e.at[huv_v.at[n]],
                              rows2_v.at[sel, pl.ds(0, L)],
                              sems.at[sel]).wait()

        @pl.when(n + 1 < NPW)
        def _():
            pltpu.async_copy(v2e.at[huv_v.at[n + 1]],
                             rows2_v.at[nsel, pl.ds(0, L)],
                             sems.at[nsel])

        selv = _splat_i(0) + sel
        rbase = n * L

        def mlp(base_list):
            nb = len(base_list)
            xs, rvs = [], []
            for base in base_list:
                idxs = base + _iota16()
                xs.append([plsc.load_gather(rows2_v, [selv, idxs, _splat_i(k)])
                           for k in range(D)])
                rvs.append(rv1_v[pl.ds(rbase + base, 16)])
            hs = [[] for _ in range(nb)]
            for d in range(D):
                accs = [plsc.load_gather(c1r_v, [rvs[e], _splat_i(d)])
                        for e in range(nb)]
                for k in range(D):
                    w = w1e_v[d, k]
                    accs = [accs[e] + w * xs[e][k] for e in range(nb)]
                for e in range(nb):
                    hs[e].append(jnp.maximum(accs[e], 0.0))
            os_ = [[] for _ in range(nb)]
            for d in range(D):
                b2 = w2be_v[d]
                accs = [b2 for _ in range(nb)]
                for k in range(D):
                    w = w2e_v[d, k]
                    accs = [accs[e] + w * hs[e][k] for e in range(nb)]
                for e in range(nb):
                    os_[e].append(jnp.maximum(accs[e], 0.0))
            t1 = [[] for _ in range(nb)]
            for d in range(D):
                cd = plsc.load_gather(cvec_v, [_splat_i(n), _splat_i(d)])
                accs = [cd for _ in range(nb)]
                for k in range(D):
                    w = a1e_v[d, k]
                    accs = [accs[e] + w * os_[e][k] for e in range(nb)]
                for e in range(nb):
                    t1[e].append(jnp.maximum(accs[e], 0.0))
            t2 = [[] for _ in range(nb)]
            for d in range(D):
                b2a = a2be_v[d]
                accs = [b2a for _ in range(nb)]
                for k in range(D):
                    w = a2e_v[d, k]
                    accs = [accs[e] + w * t1[e][k] for e in range(nb)]
                for e in range(nb):
                    t2[e].append(jnp.maximum(accs[e], 0.0))
            lgs = []
            for e in range(nb):
                lg = a3e_v[D]
                for d in range(D):
                    lg = lg + a3e_v[d] * t2[e][d]
                lgs.append(lg)
            return os_, lgs

        def pair_body(j, m):
            b0 = 3 * j * 16
            os_, lgs = mlp([b0, b0 + 16, b0 + 32])
            for e in range(3):
                for d in range(D):
                    ot_v[d, pl.ds(b0 + e * 16, 16)] = os_[e][d]
                ot_v[D, pl.ds(b0 + e * 16, 16)] = lgs[e]
                m = jnp.maximum(m, lgs[e])
            return m

        m = lax.fori_loop(0, NGROUP, pair_body,
                          jnp.full((16,), -1e30, _f32), unroll=False)

        os_, lgs = mlp([12 * 16])
        lg12 = jnp.where((12 * 16 + _iota16()) < L, lgs[0], -1e30)
        for d in range(D):
            ot_v[d, pl.ds(12 * 16, 16)] = os_[0][d]
        m = jnp.maximum(m, lg12)
        mm = jnp.max(m)

        es = [jnp.exp(ot_v[D, pl.ds(j * 16, 16)] - mm)
              for j in range(NBATCH - 1)]
        es.append(jnp.exp(lg12 - mm))
        svec = es[0]
        for j in range(1, NBATCH):
            svec = svec + es[j]
        inv = (jnp.full((16,), 1.0, _f32)
               / jnp.broadcast_to(jnp.sum(svec), (16,)))
        for d in range(D):
            acc = es[0] * ot_v[d, pl.ds(0, 16)]
            for j in range(1, NBATCH):
                acc = acc + es[j] * ot_v[d, pl.ds(j * 16, 16)]
            tb_v[d] = acc
        res = jnp.zeros((16,), _f32)
        for l in range(16):
            res = res + plsc.load_gather(tb_v, [_iota16(), _splat_i(l)])
        outb_v[n] = res * inv
        return 0

    lax.fori_loop(0, NPW, node_body, 0, unroll=False)
    pltpu.sync_copy(outb_v, out.at[pl.ds(base_node, NPW)])


@functools.partial(jax.jit, static_argnames=("interpret",))
def _run(nodes, history_uv, history_rf, v2e_w, u2e_w,
         w1e, c1r, w2e, w2be, a1e, a1uc, a1b, a2e, a2be, a3e,
         interpret=False):
    mesh = plsc.VectorSubcoreMesh(core_axis_name="c", subcore_axis_name="s",
                                  num_cores=NC, num_subcores=NS)
    k = pl.kernel(
        _sc_body,
        out_type=jax.ShapeDtypeStruct((B, D), _f32),
        mesh=mesh,
        compiler_params=pltpu.CompilerParams(needs_layout_passes=False,
                                             use_tc_tiling_on_sc=False),
        scratch_types=[
            pltpu.VMEM((NPW, L), _i32),
            pltpu.VMEM((NPW * L + 8,), _i32),
            pltpu.VMEM((NPW,), _i32),
            pltpu.VMEM((2, LP, D), _f32),
            pltpu.VMEM((NPW, D), _f32),
            pltpu.VMEM((D, D, 16), _f32),
            pltpu.VMEM((5, D), _f32),
            pltpu.VMEM((D, D, 16), _f32),
            pltpu.VMEM((D, 16), _f32),
            pltpu.VMEM((D, D, 16), _f32),
            pltpu.VMEM((D, D), _f32),
            pltpu.VMEM((D,), _f32),
            pltpu.VMEM((D, D, 16), _f32),
            pltpu.VMEM((D, 16), _f32),
            pltpu.VMEM((D + 1, 16), _f32),
            pltpu.VMEM((D + 1, LP), _f32),
            pltpu.VMEM((NPW, D), _f32),
            pltpu.VMEM((D, 16), _f32),
            pltpu.VMEM((NPW, D), _f32),
            pltpu.SemaphoreType.DMA((2,)),
            pltpu.SemaphoreType.DMA,
        ],
        interpret=interpret,
    )
    return k(history_uv, history_rf, nodes, v2e_w, u2e_w,
             w1e, c1r, w2e, w2be, a1e, a1uc, a1b, a2e, a2be, a3e)


def _prep(nodes, history_uv, history_r, v2e_w, u2e_w, r2e_w,
          w_r1_W, w_r1_b, w_r2_W, w_r2_b,
          att1_W, att1_b, att2_W, att2_b, att3_W, att3_b):
    splat = lambda M: jnp.broadcast_to(M[:, :, None], (D, D, 16))
    splat_b = lambda v: jnp.broadcast_to(v[:, None], (D, 16))
    w1e = splat(w_r1_W[:, :D])
    c1r = r2e_w @ w_r1_W[:, D:].T + w_r1_b
    w2e = splat(w_r2_W)
    w2be = splat_b(w_r2_b)
    a1e = splat(att1_W[:, :D])
    a1uc = att1_W[:, D:].T
    a2e = splat(att2_W)
    a2be = splat_b(att2_b)
    a3e = jnp.concatenate([splat_b(att3_W[0]),
                           jnp.broadcast_to(att3_b[0], (1, 16))], axis=0)
    return (nodes.astype(_i32), history_uv.astype(_i32),
            history_r.astype(_i32).reshape(-1), v2e_w, u2e_w,
            w1e, c1r, w2e, w2be, a1e, a1uc, att1_b, a2e, a2be, a3e)


def kernel(nodes, history_uv, history_r, v2e_w, u2e_w, r2e_w,
           w_r1_W, w_r1_b, w_r2_W, w_r2_b,
           att1_W, att1_b, att2_W, att2_b, att3_W, att3_b):
    args = _prep(nodes, history_uv, history_r, v2e_w, u2e_w, r2e_w,
                 w_r1_W, w_r1_b, w_r2_W, w_r2_b,
                 att1_W, att1_b, att2_W, att2_b, att3_W, att3_b)
    return _run(*args)


# speedup vs baseline: 1.6834x; 1.6834x over previous
import functools

import jax
import jax.numpy as jnp
from jax import lax
from jax.experimental import pallas as pl
from jax.experimental.pallas import tpu as pltpu
from jax.experimental.pallas import tpu_sc as plsc

B = 4096
L = 200
LP = 208
NBATCH = LP // 16
NGRP = 3
D = 16
NC, NS = 2, 16
NW = NC * NS
NPW = B // NW

_f32 = jnp.float32
_i32 = jnp.int32
_bf16 = jnp.bfloat16


def _iota16():
    return lax.iota(_i32, 16)


def _splat_i(v):
    return jnp.full((16,), v, _i32)


def _sc_body(hist_uv, hist_rf, nodes, v2e, u2e,
             w1e, c1r, w2e, w2be, a1e, a1uc, a1b, a2e, a2be, a3e,
             w1eb, w2eb, w2beb, a1eb, a2eb, a2beb, a3eb,
             out,
             huv_v, rv1_v, nidx_v, rows2_v, uvrows_v,
             w1e_v, c1r_v, w2e_v, w2be_v, a1e_v, a1uc_v, a1b_v,
             a2e_v, a2be_v, a3e_v,
             w1eb_v, w2eb_v, w2beb_v, a1eb_v, a2eb_v, a2beb_v, a3eb_v,
             ot_v, cvec_v, tb_v, outb_v, sems, sem):
    wid = lax.axis_index("s") * NC + lax.axis_index("c")
    base_node = wid * NPW

    pltpu.sync_copy(w1e, w1e_v)
    pltpu.sync_copy(c1r, c1r_v)
    pltpu.sync_copy(w2e, w2e_v)
    pltpu.sync_copy(w2be, w2be_v)
    pltpu.sync_copy(a1e, a1e_v)
    pltpu.sync_copy(a1uc, a1uc_v)
    pltpu.sync_copy(a1b, a1b_v)
    pltpu.sync_copy(a2e, a2e_v)
    pltpu.sync_copy(a2be, a2be_v)
    pltpu.sync_copy(a3e, a3e_v)
    pltpu.sync_copy(w1eb, w1eb_v)
    pltpu.sync_copy(w2eb, w2eb_v)
    pltpu.sync_copy(w2beb, w2beb_v)
    pltpu.sync_copy(a1eb, a1eb_v)
    pltpu.sync_copy(a2eb, a2eb_v)
    pltpu.sync_copy(a2beb, a2beb_v)
    pltpu.sync_copy(a3eb, a3eb_v)

    rv1_v[pl.ds(NPW * L - 8, 16)] = jnp.zeros((16,), _i32)
    pltpu.sync_copy(hist_uv.at[pl.ds(base_node, NPW)], huv_v)
    pltpu.sync_copy(hist_rf.at[pl.ds(base_node * L, NPW * L)],
                    rv1_v.at[pl.ds(0, NPW * L)])

    pltpu.sync_copy(nodes.at[pl.ds(base_node, NPW)], nidx_v)
    pltpu.async_copy(u2e.at[nidx_v], uvrows_v, sem).wait()

    def catt_body(n, _):
        catt = a1b_v[...]
        for k in range(D):
            uk = plsc.load_gather(uvrows_v, [_splat_i(n), _splat_i(k)])
            catt = catt + uk * a1uc_v[k]
        cvec_v[n] = catt
        return 0

    lax.fori_loop(0, NPW, catt_body, 0, unroll=False)

    for s in range(2):
        for r in range(L, LP):
            rows2_v[s, r] = jnp.zeros((16,), _f32)

    pltpu.async_copy(v2e.at[huv_v.at[0]], rows2_v.at[0, pl.ds(0, L)],
                     sems.at[0])

    def node_body(n, _):
        sel = lax.rem(n, 2)
        nsel = 1 - sel
        pltpu.make_async_copy(v2e.at[huv_v.at[n]],
                              rows2_v.at[sel, pl.ds(0, L)],
                              sems.at[sel]).wait()

        @pl.when(n + 1 < NPW)
        def _():
            pltpu.async_copy(v2e.at[huv_v.at[n + 1]],
                             rows2_v.at[nsel, pl.ds(0, L)],
                             sems.at[nsel])

        selv = _splat_i(0) + sel
        rbase = n * L

        def mlp(base_list):
            nb = len(base_list)
            xs, rvs = [], []
            for base in base_list:
                idxs = base + _iota16()
                xs.append([plsc.load_gather(rows2_v, [selv, idxs, _splat_i(k)])
                           for k in range(D)])
                rvs.append(rv1_v[pl.ds(rbase + base, 16)])
            hs = [[] for _ in range(nb)]
            for d in range(D):
                accs = [plsc.load_gather(c1r_v, [rvs[e], _splat_i(d)])
                        for e in range(nb)]
                for k in range(D):
                    w = w1e_v[d, k]
                    accs = [accs[e] + w * xs[e][k] for e in range(nb)]
                for e in range(nb):
                    hs[e].append(jnp.maximum(accs[e], 0.0))
            os_ = [[] for _ in range(nb)]
            for d in range(D):
                b2 = w2be_v[d]
                accs = [b2 for _ in range(nb)]
                for k in range(D):
                    w = w2e_v[d, k]
                    accs = [accs[e] + w * hs[e][k] for e in range(nb)]
                for e in range(nb):
                    os_[e].append(jnp.maximum(accs[e], 0.0))
            t1 = [[] for _ in range(nb)]
            for d in range(D):
                cd = plsc.load_gather(cvec_v, [_splat_i(n), _splat_i(d)])
                accs = [cd for _ in range(nb)]
                for k in range(D):
                    w = a1e_v[d, k]
                    accs = [accs[e] + w * os_[e][k] for e in range(nb)]
                for e in range(nb):
                    t1[e].append(jnp.maximum(accs[e], 0.0))
            t2 = [[] for _ in range(nb)]
            for d in range(D):
                b2a = a2be_v[d]
                accs = [b2a for _ in range(nb)]
                for k in range(D):
                    w = a2e_v[d, k]
                    accs = [accs[e] + w * t1[e][k] for e in range(nb)]
                for e in range(nb):
                    t2[e].append(jnp.maximum(accs[e], 0.0))
            lgs = []
            for e in range(nb):
                lg = a3e_v[D]
                for d in range(D):
                    lg = lg + a3e_v[d] * t2[e][d]
                lgs.append(lg)
            return os_, lgs

        def mlp4(b0):
            xs, rvs = [], []
            for e in range(4):
                base = b0 + e * 16
                idxs = base + _iota16()
                xs.append([plsc.load_gather(rows2_v, [selv, idxs, _splat_i(k)])
                           for k in range(D)])
                rvs.append(rv1_v[pl.ds(rbase + base, 16)])
            xp = [[plsc.pack(xs[2 * p][k], xs[2 * p + 1][k],
                             format=plsc.PackFormat.INTERLEAVED)
                   for k in range(D)] for p in range(2)]
            hs = [[] for _ in range(2)]
            for d in range(D):
                cs = [plsc.load_gather(c1r_v, [rvs[e], _splat_i(d)])
                      for e in range(4)]
                accs = [plsc.pack(cs[0], cs[1],
                                  format=plsc.PackFormat.INTERLEAVED),
                        plsc.pack(cs[2], cs[3],
                                  format=plsc.PackFormat.INTERLEAVED)]
                for k in range(D):
                    w = w1eb_v[d, k]
                    accs = [accs[p] + w * xp[p][k] for p in range(2)]
                for p in range(2):
                    hs[p].append(jnp.maximum(accs[p], 0))
            os_ = [[] for _ in range(2)]
            for d in range(D):
                b2 = w2beb_v[d]
                accs = [b2, b2]
                for k in range(D):
                    w = w2eb_v[d, k]
                    accs = [accs[p] + w * hs[p][k] for p in range(2)]
                for p in range(2):
                    os_[p].append(jnp.maximum(accs[p], 0))
            t1 = [[] for _ in range(2)]
            for d in range(D):
                cd = plsc.load_gather(cvec_v, [_splat_i(n), _splat_i(d)])
                cdp = plsc.pack(cd, cd, format=plsc.PackFormat.INTERLEAVED)
                accs = [cdp, cdp]
                for k in range(D):
                    w = a1eb_v[d, k]
                    accs = [accs[p] + w * os_[p][k] for p in range(2)]
                for p in range(2):
                    t1[p].append(jnp.maximum(accs[p], 0))
            t2 = [[] for _ in range(2)]
            for d in range(D):
                b2a = a2beb_v[d]
                accs = [b2a, b2a]
                for k in range(D):
                    w = a2eb_v[d, k]
                    accs = [accs[p] + w * t1[p][k] for p in range(2)]
                for p in range(2):
                    t2[p].append(jnp.maximum(accs[p], 0))
            lgs = []
            for p in range(2):
                lg = a3eb_v[D]
                for d in range(D):
                    lg = lg + a3eb_v[d] * t2[p][d]
                la, lb = plsc.unpack(lg, format=plsc.PackFormat.INTERLEAVED)
                lgs += [la.astype(_f32), lb.astype(_f32)]
            ous = [[] for _ in range(4)]
            for d in range(D):
                for p in range(2):
                    oa, ob = plsc.unpack(os_[p][d],
                                         format=plsc.PackFormat.INTERLEAVED)
                    ous[2 * p].append(oa.astype(_f32))
                    ous[2 * p + 1].append(ob.astype(_f32))
            return ous, lgs

        def group_body(j, m):
            b0 = 4 * j * 16
            os_, lgs = mlp4(b0)
            for e in range(4):
                for d in range(D):
                    ot_v[d, pl.ds(b0 + e * 16, 16)] = os_[e][d]
                ot_v[D, pl.ds(b0 + e * 16, 16)] = lgs[e]
                m = jnp.maximum(m, lgs[e])
            return m

        m = lax.fori_loop(0, NGRP, group_body,
                          jnp.full((16,), -1e30, _f32), unroll=False)

        os_, lgs = mlp([12 * 16])
        lg12 = jnp.where((12 * 16 + _iota16()) < L, lgs[0], -1e30)
        for d in range(D):
            ot_v[d, pl.ds(12 * 16, 16)] = os_[0][d]
        m = jnp.maximum(m, lg12)
        mm = jnp.max(m)

        es = [jnp.exp(ot_v[D, pl.ds(j * 16, 16)] - mm)
              for j in range(NBATCH - 1)]
        es.append(jnp.exp(lg12 - mm))
        svec = es[0]
        for j in range(1, NBATCH):
            svec = svec + es[j]
        inv = (jnp.full((16,), 1.0, _f32)
               / jnp.broadcast_to(jnp.sum(svec), (16,)))
        for d in range(D):
            acc = es[0] * ot_v[d, pl.ds(0, 16)]
            for j in range(1, NBATCH):
                acc = acc + es[j] * ot_v[d, pl.ds(j * 16, 16)]
            tb_v[d] = acc
        res = jnp.zeros((16,), _f32)
        for l in range(16):
            res = res + plsc.load_gather(tb_v, [_iota16(), _splat_i(l)])
        outb_v[n] = res * inv
        return 0

    lax.fori_loop(0, NPW, node_body, 0, unroll=False)
    pltpu.sync_copy(outb_v, out.at[pl.ds(base_node, NPW)])


@functools.partial(jax.jit, static_argnames=("interpret",))
def _run(nodes, history_uv, history_rf, v2e_w, u2e_w,
         w1e, c1r, w2e, w2be, a1e, a1uc, a1b, a2e, a2be, a3e,
         w1eb, w2eb, w2beb, a1eb, a2eb, a2beb, a3eb,
         interpret=False):
    mesh = plsc.VectorSubcoreMesh(core_axis_name="c", subcore_axis_name="s",
                                  num_cores=NC, num_subcores=NS)
    k = pl.kernel(
        _sc_body,
        out_type=jax.ShapeDtypeStruct((B, D), _f32),
        mesh=mesh,
        compiler_params=pltpu.CompilerParams(needs_layout_passes=False,
                                             use_tc_tiling_on_sc=False),
        scratch_types=[
            pltpu.VMEM((NPW, L), _i32),
            pltpu.VMEM((NPW * L + 8,), _i32),
            pltpu.VMEM((NPW,), _i32),
            pltpu.VMEM((2, LP, D), _f32),
            pltpu.VMEM((NPW, D), _f32),
            pltpu.VMEM((D, D, 16), _f32),
            pltpu.VMEM((5, D), _f32),
            pltpu.VMEM((D, D, 16), _f32),
            pltpu.VMEM((D, 16), _f32),
            pltpu.VMEM((D, D, 16), _f32),
            pltpu.VMEM((D, D), _f32),
            pltpu.VMEM((D,), _f32),
            pltpu.VMEM((D, D, 16), _f32),
            pltpu.VMEM((D, 16), _f32),
            pltpu.VMEM((D + 1, 16), _f32),
            pltpu.VMEM((D, D, 32), _bf16),
            pltpu.VMEM((D, D, 32), _bf16),
            pltpu.VMEM((D, 32), _bf16),
            pltpu.VMEM((D, D, 32), _bf16),
            pltpu.VMEM((D, D, 32), _bf16),
            pltpu.VMEM((D, 32), _bf16),
            pltpu.VMEM((D + 1, 32), _bf16),
            pltpu.VMEM((D + 1, LP), _f32),
            pltpu.VMEM((NPW, D), _f32),
            pltpu.VMEM((D, 16), _f32),
            pltpu.VMEM((NPW, D), _f32),
            pltpu.SemaphoreType.DMA((2,)),
            pltpu.SemaphoreType.DMA,
        ],
        interpret=interpret,
    )
    return k(history_uv, history_rf, nodes, v2e_w, u2e_w,
             w1e, c1r, w2e, w2be, a1e, a1uc, a1b, a2e, a2be, a3e,
             w1eb, w2eb, w2beb, a1eb, a2eb, a2beb, a3eb)


def _prep(nodes, history_uv, history_r, v2e_w, u2e_w, r2e_w,
          w_r1_W, w_r1_b, w_r2_W, w_r2_b,
          att1_W, att1_b, att2_W, att2_b, att3_W, att3_b):
    splat = lambda M: jnp.broadcast_to(M[:, :, None], (D, D, 16))
    splat_b = lambda v: jnp.broadcast_to(v[:, None], (D, 16))
    w1e = splat(w_r1_W[:, :D])
    c1r = r2e_w @ w_r1_W[:, D:].T + w_r1_b
    w2e = splat(w_r2_W)
    w2be = splat_b(w_r2_b)
    a1e = splat(att1_W[:, :D])
    a1uc = att1_W[:, D:].T
    a2e = splat(att2_W)
    a2be = splat_b(att2_b)
    a3e = jnp.concatenate([splat_b(att3_W[0]),
                           jnp.broadcast_to(att3_b[0], (1, 16))], axis=0)
    splat32 = lambda M: jnp.broadcast_to(
        M.astype(_bf16)[:, :, None], (D, D, 32))
    splat32_b = lambda v: jnp.broadcast_to(v.astype(_bf16)[:, None], (D, 32))
    w1eb = splat32(w_r1_W[:, :D])
    w2eb = splat32(w_r2_W)
    w2beb = splat32_b(w_r2_b)
    a1eb = splat32(att1_W[:, :D])
    a2eb = splat32(att2_W)
    a2beb = splat32_b(att2_b)
    a3eb = jnp.concatenate(
        [splat32_b(att3_W[0]),
         jnp.broadcast_to(att3_b.astype(_bf16)[0], (1, 32))], axis=0)
    return (nodes.astype(_i32), history_uv.astype(_i32),
            history_r.astype(_i32).reshape(-1), v2e_w, u2e_w,
            w1e, c1r, w2e, w2be, a1e, a1uc, att1_b, a2e, a2be, a3e,
            w1eb, w2eb, w2beb, a1eb, a2eb, a2beb, a3eb)


def kernel(nodes, history_uv, history_r, v2e_w, u2e_w, r2e_w,
           w_r1_W, w_r1_b, w_r2_W, w_r2_b,
           att1_W, att1_b, att2_W, att2_b, att3_W, att3_b):
    args = _prep(nodes, history_uv, history_r, v2e_w, u2e_w, r2e_w,
                 w_r1_W, w_r1_b, w_r2_W, w_r2_b,
                 att1_W, att1_b, att2_W, att2_b, att3_W, att3_b)
    return _run(*args)
